# baseline (speedup 1.0000x reference)
import dataclasses
import functools

import jax
import jax.numpy as jnp
from jax import lax
from jax.experimental import pallas as pl
from jax.experimental.pallas import tpu as pltpu
from jax.experimental.pallas import tpu_sc as plsc

N = 10000
E = 320000
D = 128
NG = 64
G8 = 16

NC = 2
NS = 16
L = 16
NW = NC * NS
C = 80
ZR = 624
FLAT = 40960

BE = 5000
BN = 2000

SPLITS = ((0, 163840, 5120), (163840, 156160, 2560))


def _sc_compiler_params():
    cp = pltpu.CompilerParams()
    if "needs_layout_passes" in pltpu.CompilerParams.__dataclass_fields__:
        cp = dataclasses.replace(cp, needs_layout_passes=False)
    return cp


def _silu(x):
    return x * jax.nn.sigmoid(x)


def _ln(x, w, b):
    m = jnp.mean(x, axis=-1, keepdims=True)
    v = jnp.mean((x - m) ** 2, axis=-1, keepdims=True)
    return (x - m) / jnp.sqrt(v + 1e-5) * w + b


def _node_stats_body(b2_ref, br_ref, x8_ref, e3_ref, xn8_ref):
    x8 = x8_ref[...]
    norm = jnp.sqrt(jnp.sum(x8 * x8, axis=-1, keepdims=True))
    gcol = lax.broadcasted_iota(jnp.int32, (NG, 1), 0)
    ohT = (gcol == br_ref[...]).astype(jnp.float32)
    nc = jnp.concatenate([norm, jnp.ones_like(norm)], axis=1)
    sums = jnp.dot(ohT, nc, preferred_element_type=jnp.float32)
    mg = sums[:, 0:1] / jnp.maximum(sums[:, 1:2], 1.0)
    grow = lax.broadcasted_iota(jnp.int32, (1, NG), 1)
    oh = (b2_ref[...] == grow).astype(jnp.float32)
    mn = jnp.dot(oh, mg, preferred_element_type=jnp.float32)
    xn8_ref[...] = e3_ref[0, 0] * x8 / (mn + 1e-5)


def _node_stats(batch2, batchr, x8, e3w):
    return pl.pallas_call(
        _node_stats_body,
        out_shape=jax.ShapeDtypeStruct((N, G8), jnp.float32),
    )(batch2, batchr, x8, e3w)


def _node_feat_body(h_ref, qw1_ref, qw2_ref, qnw_ref, qnb_ref,
                    gw1_ref, gb1_ref, gw2r_ref, gb2_ref, kvw1h_ref,
                    gate_ref, qn_ref, hkv_ref):
    h = h_ref[...]
    q = jnp.dot(_silu(jnp.dot(h, qw1_ref[...],
                              preferred_element_type=jnp.float32)),
                qw2_ref[...], preferred_element_type=jnp.float32)
    qn_ref[...] = _ln(q, qnw_ref[...], qnb_ref[...])
    g1 = _silu(jnp.dot(h, gw1_ref[...],
                       preferred_element_type=jnp.float32) + gb1_ref[...])
    gate_ref[...] = jax.nn.sigmoid(
        jnp.sum(g1 * gw2r_ref[...], axis=-1, keepdims=True) + gb2_ref[...])
    hkv_ref[...] = jnp.dot(h, kvw1h_ref[...],
                           preferred_element_type=jnp.float32)


def _node_feats(h, qw1, qw2, qnw, qnb, gw1, gb1, gw2r, gb2, kvw1h):
    nblk = N // BN
    row_spec = lambda shp: pl.BlockSpec(shp, lambda i: (i, 0))
    full = lambda shp: pl.BlockSpec(shp, lambda i: (0, 0))
    return pl.pallas_call(
        _node_feat_body,
        grid=(nblk,),
        in_specs=[row_spec((BN, D)),
                  full((D, D)), full((D, D)), full((1, D)), full((1, D)),
                  full((D, D)), full((1, D)), full((1, D)), full((1, 1)),
                  full((D, 2 * D))],
        out_specs=[row_spec((BN, 1)), row_spec((BN, D)),
                   row_spec((BN, 2 * D))],
        out_shape=[jax.ShapeDtypeStruct((N, 1), jnp.float32),
                   jax.ShapeDtypeStruct((N, D), jnp.float32),
                   jax.ShapeDtypeStruct((N, 2 * D), jnp.float32)],
    )(h, qw1, qw2, qnw, qnb, gw1, gb1, gw2r, gb2, kvw1h)


def _pb_body(ze_ref, pbw1_ref, pbb1_ref, pbw2r_ref, pbb2_ref, b_ref):
    zew = _silu(jnp.dot(ze_ref[...], pbw1_ref[...],
                        preferred_element_type=jnp.float32) + pbb1_ref[...])
    b_ref[...] = jnp.sum(zew * pbw2r_ref[...], axis=-1,
                         keepdims=True) + pbb2_ref[...]


def _pb_compute(ze, pbw1, pbb1, pbw2r, pbb2):
    nblk = E // BE
    row_spec = lambda shp: pl.BlockSpec(shp, lambda i: (i, 0))
    full = lambda shp: pl.BlockSpec(shp, lambda i: (0, 0))
    return pl.pallas_call(
        _pb_body,
        grid=(nblk,),
        in_specs=[row_spec((BE, D)), full((D, 4 * D)), full((1, 4 * D)),
                  full((1, 4 * D)), full((1, 1))],
        out_specs=row_spec((BE, 1)),
        out_shape=jax.ShapeDtypeStruct((E, 1), jnp.float32),
    )(ze, pbw1, pbb1, pbw2r, pbb2)


def _sc_gather_body(lo, nchunk, src_hbm, dst_hbm, hkv_hbm, qn_hbm,
                    tab4_hbm, hkvs_out, qnd_out, rel4_out,
                    sidx, didx, hbuf0, hbuf1, qbuf0, qbuf1, r4b0, r4b1,
                    tab4, s1, s2, s3, s4, so0, so1):
    ew = nchunk * C
    wid = lax.axis_index("s") * NC + lax.axis_index("c")
    base = wid * ew
    pltpu.sync_copy(tab4_hbm, tab4)
    pltpu.sync_copy(src_hbm.at[pl.ds(lo + base, ew)], sidx)
    pltpu.sync_copy(dst_hbm.at[pl.ds(lo + base, ew)], didx)

    def reg_work(g, r4b):
        @pl.loop(0, C // L)
        def _(j):
            sl = pl.ds(g * C + j * L, L)
            sv4 = sidx[sl] * 4
            dv4 = didx[sl] * 4
            ex = plsc.load_gather(tab4, [sv4]) - plsc.load_gather(tab4, [dv4])
            ey = (plsc.load_gather(tab4, [sv4 + 1])
                  - plsc.load_gather(tab4, [dv4 + 1]))
            ez = (plsc.load_gather(tab4, [sv4 + 2])
                  - plsc.load_gather(tab4, [dv4 + 2]))
            gg = plsc.load_gather(tab4, [sv4 + 3])
            pos = (lax.iota(jnp.int32, L) + j * L) * 4
            plsc.store_scatter(r4b, [pos], ex)
            plsc.store_scatter(r4b, [pos + 1], ey)
            plsc.store_scatter(r4b, [pos + 2], ez)
            plsc.store_scatter(r4b, [pos + 3], gg)

    def fire_gathers(g, hbuf, qbuf, sa, sb):
        ca = pltpu.async_copy(hkv_hbm.at[sidx.at[pl.ds(g * C, C)]], hbuf, sa)
        cb = pltpu.async_copy(qn_hbm.at[didx.at[pl.ds(g * C, C)]], qbuf, sb)
        return ca, cb

    def fire_outs(g, hbuf, qbuf, r4b, so):
        off = base + g * C
        o1 = pltpu.async_copy(hbuf, hkvs_out.at[pl.ds(off, C)], so)
        o2 = pltpu.async_copy(qbuf, qnd_out.at[pl.ds(off, C)], so)
        o3 = pltpu.async_copy(r4b, rel4_out.at[pl.ds(off * 4, 4 * C)], so)
        return o1, o2, o3

    @pl.loop(0, nchunk // 2)
    def _(p):
        g0 = 2 * p
        g1 = g0 + 1
        c1, c2 = fire_gathers(g0, hbuf0, qbuf0, s1, s2)
        c3, c4 = fire_gathers(g1, hbuf1, qbuf1, s3, s4)
        reg_work(g0, r4b0)
        reg_work(g1, r4b1)
        c1.wait()
        c2.wait()
        outs0 = fire_outs(g0, hbuf0, qbuf0, r4b0, so0)
        c3.wait()
        c4.wait()
        outs1 = fire_outs(g1, hbuf1, qbuf1, r4b1, so1)
        for o in outs0 + outs1:
            o.wait()

    if nchunk % 2:
        g = nchunk - 1
        c1, c2 = fire_gathers(g, hbuf0, qbuf0, s1, s2)
        reg_work(g, r4b0)
        c1.wait()
        c2.wait()
        for o in fire_outs(g, hbuf0, qbuf0, r4b0, so0):
            o.wait()


def _sc_gather(src, dst, hkv, qn, tab4, lo, ne):
    ew = ne // NW
    assert ew % C == 0
    mesh = plsc.VectorSubcoreMesh(core_axis_name="c", subcore_axis_name="s",
                                  num_cores=NC, num_subcores=NS)
    f = functools.partial(
        pl.kernel,
        out_type=(jax.ShapeDtypeStruct((ne, 2 * D), jnp.float32),
                  jax.ShapeDtypeStruct((ne, D), jnp.float32),
                  jax.ShapeDtypeStruct((4 * ne,), jnp.float32)),
        mesh=mesh,
        scratch_types=[pltpu.VMEM((ew,), jnp.int32),
                       pltpu.VMEM((ew,), jnp.int32),
                       pltpu.VMEM((C, 2 * D), jnp.float32),
                       pltpu.VMEM((C, 2 * D), jnp.float32),
                       pltpu.VMEM((C, D), jnp.float32),
                       pltpu.VMEM((C, D), jnp.float32),
                       pltpu.VMEM((4 * C,), jnp.float32),
                       pltpu.VMEM((4 * C,), jnp.float32),
                       pltpu.VMEM((4 * N,), jnp.float32),
                       pltpu.SemaphoreType.DMA,
                       pltpu.SemaphoreType.DMA,
                       pltpu.SemaphoreType.DMA,
                       pltpu.SemaphoreType.DMA,
                       pltpu.SemaphoreType.DMA,
                       pltpu.SemaphoreType.DMA],
        compiler_params=_sc_compiler_params(),
    )(functools.partial(_sc_gather_body, lo, ew // C))
    return f(src, dst, hkv, qn, tab4)


def _edge_body(hkvs_ref, qnd_ref, rel4_ref, b_ref,
               kvw10_ref, kvw2_ref, knw_ref, knb_ref,
               att_ref, ax4_ref):
    rel4 = rel4_ref[...]
    rx = rel4[:, 0:1]
    ry = rel4[:, 1:2]
    rz = rel4[:, 2:3]
    gs = rel4[:, 3:4]
    rd = rx * rx + ry * ry + rz * rz
    inv = 1.0 / (1.0 + jnp.sqrt(rd + 1e-8))
    pre = hkvs_ref[...] + rd * kvw10_ref[...]
    kv = jnp.dot(_silu(pre), kvw2_ref[...],
                 preferred_element_type=jnp.float32)
    k = _ln(kv[:, :D], knw_ref[...], knb_ref[...])
    v = kv[:, D:]
    scores = jnp.sum(qnd_ref[...] * k, axis=-1, keepdims=True) * (
        1.0 / jnp.sqrt(jnp.float32(D))) + b_ref[...]
    alpha = gs * scores
    att_ref[...] = alpha * v
    w = alpha * inv
    ax4_ref[...] = jnp.concatenate(
        [w * rx, w * ry, w * rz, jnp.zeros_like(w)], axis=1)


def _edge_compute(hkvs, qnd, rel4, b, kvw10, kvw2, knw, knb, lo, ne, be):
    nblk = ne // be
    assert ne % be == 0 and be % 8 == 0 and lo % be == 0
    ob = lo // be
    row_spec = lambda shp: pl.BlockSpec(shp, lambda i: (i, 0))
    off_spec = lambda shp: pl.BlockSpec(shp, lambda i: (i + ob, 0))
    full = lambda shp: pl.BlockSpec(shp, lambda i: (0, 0))
    return pl.pallas_call(
        _edge_body,
        grid=(nblk,),
        in_specs=[row_spec((be, 2 * D)), row_spec((be, D)),
                  row_spec((be, 4)), off_spec((be, 1)),
                  full((1, 2 * D)), full((2 * D, 2 * D)),
                  full((1, D)), full((1, D))],
        out_specs=[row_spec((be, D)), row_spec((be, 4))],
        out_shape=[jax.ShapeDtypeStruct((ne, D), jnp.float32),
                   jax.ShapeDtypeStruct((ne, 4), jnp.float32)],
    )(hkvs, qnd, rel4, b, kvw10, kvw2, knw, knb)


def _sc_scatter_body(lo, nchunk, dst_hbm, att_hbm, ax4_hbm, z128_hbm, zflat_hbm,
                     a_out, axf_out,
                     didx0, didx1, ix0, iy0, iz0, ix1, iy1, iz1,
                     attb0, attb1, a4b0, a4b1, vx0, vy0, vz0, vx1, vy1, vz1,
                     a_sh, axf_sh, s1, s2, s3, s4, sw0, sw1):
    ew = nchunk * C
    cid = lax.axis_index("c")
    sid = lax.axis_index("s")
    wid = sid * NC + cid
    pltpu.sync_copy(z128_hbm, a_sh.at[pl.ds(sid * ZR, ZR)])

    @pl.when(sid == 0)
    def _():
        pltpu.sync_copy(z128_hbm.at[pl.ds(0, N - NS * ZR)],
                        a_sh.at[pl.ds(NS * ZR, N - NS * ZR)])
        pltpu.sync_copy(zflat_hbm, axf_sh)

    plsc.subcore_barrier()
    base = wid * ew

    def fire_loads(g, didx, attb, a4b, sa, sb):
        off = base + g * C
        pltpu.sync_copy(dst_hbm.at[pl.ds(lo + off, C)], didx)
        ca = pltpu.async_copy(att_hbm.at[pl.ds(off, C)], attb, sa)
        cb = pltpu.async_copy(ax4_hbm.at[pl.ds(off * 4, 4 * C)], a4b, sb)
        return ca, cb

    def reg_work(didx, a4b, ix, iy, iz, vx, vy, vz):
        @pl.loop(0, C // L)
        def _(j):
            sl = pl.ds(j * L, L)
            dv4 = didx[sl] * 4
            ix[sl] = dv4
            iy[sl] = dv4 + 1
            iz[sl] = dv4 + 2
            pos = (lax.iota(jnp.int32, L) + j * L) * 4
            vx[sl] = plsc.load_gather(a4b, [pos])
            vy[sl] = plsc.load_gather(a4b, [pos + 1])
            vz[sl] = plsc.load_gather(a4b, [pos + 2])

    def fire_scatters(didx, attb, ix, iy, iz, vx, vy, vz, sw):
        o1 = pltpu.async_copy(attb, a_sh.at[didx], sw, add=True)
        o2 = pltpu.async_copy(vx, axf_sh.at[ix], sw, add=True)
        o3 = pltpu.async_copy(vy, axf_sh.at[iy], sw, add=True)
        o4 = pltpu.async_copy(vz, axf_sh.at[iz], sw, add=True)
        return o1, o2, o3, o4

    @pl.loop(0, nchunk // 2)
    def _(p):
        g0 = 2 * p
        g1 = g0 + 1
        c1, c2 = fire_loads(g0, didx0, attb0, a4b0, s1, s2)
        c3, c4 = fire_loads(g1, didx1, attb1, a4b1, s3, s4)
        c1.wait()
        c2.wait()
        reg_work(didx0, a4b0, ix0, iy0, iz0, vx0, vy0, vz0)
        outs0 = fire_scatters(didx0, attb0, ix0, iy0, iz0, vx0, vy0, vz0, sw0)
        c3.wait()
        c4.wait()
        reg_work(didx1, a4b1, ix1, iy1, iz1, vx1, vy1, vz1)
        outs1 = fire_scatters(didx1, attb1, ix1, iy1, iz1, vx1, vy1, vz1, sw1)
        for o in outs0 + outs1:
            o.wait()

    if nchunk % 2:
        g = nchunk - 1
        c1, c2 = fire_loads(g, didx0, attb0, a4b0, s1, s2)
        c1.wait()
        c2.wait()
        reg_work(didx0, a4b0, ix0, iy0, iz0, vx0, vy0, vz0)
        for o in fire_scatters(didx0, attb0, ix0, iy0, iz0, vx0, vy0, vz0,
                               sw0):
            o.wait()

    plsc.subcore_barrier()
    row0 = cid * N + sid * ZR
    pltpu.sync_copy(a_sh.at[pl.ds(sid * ZR, ZR)], a_out.at[pl.ds(row0, ZR)])

    @pl.when(sid == 0)
    def _():
        pltpu.sync_copy(a_sh.at[pl.ds(NS * ZR, N - NS * ZR)],
                        a_out.at[pl.ds(cid * N + NS * ZR, N - NS * ZR)])
        pltpu.sync_copy(axf_sh, axf_out.at[pl.ds(cid * FLAT, FLAT)])


def _sc_scatter(dst, att, ax4, z128, zflat, lo, ne):
    ew = ne // NW
    assert ew % C == 0
    mesh = plsc.VectorSubcoreMesh(core_axis_name="c", subcore_axis_name="s",
                                  num_cores=NC, num_subcores=NS)
    cvec = lambda dt: pltpu.VMEM((C,), dt)
    f = functools.partial(
        pl.kernel,
        out_type=(jax.ShapeDtypeStruct((2 * N, D), jnp.float32),
                  jax.ShapeDtypeStruct((2 * FLAT,), jnp.float32)),
        mesh=mesh,
        scratch_types=[cvec(jnp.int32), cvec(jnp.int32),
                       cvec(jnp.int32), cvec(jnp.int32), cvec(jnp.int32),
                       cvec(jnp.int32), cvec(jnp.int32), cvec(jnp.int32),
                       pltpu.VMEM((C, D), jnp.float32),
                       pltpu.VMEM((C, D), jnp.float32),
                       pltpu.VMEM((4 * C,), jnp.float32),
                       pltpu.VMEM((4 * C,), jnp.float32),
                       cvec(jnp.float32), cvec(jnp.float32),
                       cvec(jnp.float32), cvec(jnp.float32),
                       cvec(jnp.float32), cvec(jnp.float32),
                       pltpu.VMEM_SHARED((N, D), jnp.float32),
                       pltpu.VMEM_SHARED((FLAT,), jnp.float32),
                       pltpu.SemaphoreType.DMA,
                       pltpu.SemaphoreType.DMA,
                       pltpu.SemaphoreType.DMA,
                       pltpu.SemaphoreType.DMA,
                       pltpu.SemaphoreType.DMA,
                       pltpu.SemaphoreType.DMA],
        compiler_params=_sc_compiler_params(),
    )(functools.partial(_sc_scatter_body, lo, ew // C))
    return f(dst, att, ax4, z128, zflat)


def _final_body(a2a_ref, a2b_ref, axfa_ref, axfb_ref, xn8_ref, h_ref,
                pxw1_ref, pxw2r_ref, phw1_ref, phb1_ref, phw2_ref, phb2_ref,
                xout_ref, hout_ref):
    a2 = a2a_ref[...] + a2b_ref[...]
    a = a2[0] + a2[1]
    axf = axfa_ref[...] + axfb_ref[...]
    axv = axf[0, :, 0:3] + axf[1, :, 0:3]
    phix = jnp.sum(_silu(jnp.dot(a, pxw1_ref[...],
                                 preferred_element_type=jnp.float32))
                   * pxw2r_ref[...], axis=-1, keepdims=True)
    xout_ref[...] = xn8_ref[...][:, 0:3] + phix * axv
    h = h_ref[...]
    t = _silu(jnp.dot(a * a * h, phw1_ref[...],
                      preferred_element_type=jnp.float32) + phb1_ref[...])
    hout_ref[...] = h + jnp.dot(t, phw2_ref[...],
                                preferred_element_type=jnp.float32) + phb2_ref[...]


def _final(a2a, a2b, axfa, axfb, xn8, h, pxw1, pxw2r, phw1, phb1, phw2,
           phb2):
    nblk = N // BN
    rs3 = lambda shp: pl.BlockSpec(shp, lambda i: (0, i, 0))
    rs2 = lambda shp: pl.BlockSpec(shp, lambda i: (i, 0))
    full = lambda shp: pl.BlockSpec(shp, lambda i: (0, 0))
    return pl.pallas_call(
        _final_body,
        grid=(nblk,),
        in_specs=[rs3((2, BN, D)), rs3((2, BN, D)),
                  rs3((2, BN, 4)), rs3((2, BN, 4)),
                  rs2((BN, G8)), rs2((BN, D)),
                  full((D, D)), full((1, D)), full((D, D)), full((1, D)),
                  full((D, D)), full((1, D))],
        out_specs=[rs2((BN, 3)), rs2((BN, D))],
        out_shape=[jax.ShapeDtypeStruct((N, 3), jnp.float32),
                   jax.ShapeDtypeStruct((N, D), jnp.float32)],
    )(a2a.reshape(2, N, D), a2b.reshape(2, N, D), axfa, axfb, xn8, h,
      pxw1, pxw2r, phw1, phb1, phw2, phb2)


def kernel(batch, X, H, E_idx, ZE, e3_w, kv_w1, kv_w2, q_w1, q_w2, pb_w1,
           pb_b1, pb_w2, pb_b2, gate_w1, gate_b1, gate_w2, gate_b2, phih_w1,
           phih_b1, phih_w2, phih_b2, phix_w1, phix_w2, qn_w, qn_b, kn_w,
           kn_b):
    batch2 = batch.reshape(N, 1).astype(jnp.int32)
    batchr = batch.reshape(1, N).astype(jnp.int32)
    x8 = jnp.concatenate([X, jnp.zeros((N, G8 - 3), jnp.float32)], axis=1)
    src = E_idx[0].astype(jnp.int32)
    dst = E_idx[1].astype(jnp.int32)

    xn8 = _node_stats(batch2, batchr, x8, e3_w)
    gate, qn, hkv = _node_feats(
        H, q_w1, q_w2, qn_w.reshape(1, D), qn_b.reshape(1, D),
        gate_w1, gate_b1.reshape(1, D), gate_w2.reshape(1, D),
        gate_b2.reshape(1, 1), kv_w1[1:])

    z128 = jnp.zeros((ZR, D), jnp.float32)
    zflat = jnp.zeros((FLAT,), jnp.float32)
    tab4 = jnp.concatenate([xn8[:, 0:3], gate], axis=1).reshape(4 * N)

    b_full = _pb_compute(ZE, pb_w1, pb_b1.reshape(1, 4 * D),
                         pb_w2.reshape(1, 4 * D), pb_b2.reshape(1, 1))

    gouts = [_sc_gather(src, dst, hkv, qn, tab4, lo, ne)
             for lo, ne, be in SPLITS]

    parts = []
    for (lo, ne, be), g in zip(SPLITS, gouts):
        hkvs, qnd, rel4 = g
        att, ax4 = _edge_compute(hkvs, qnd, rel4.reshape(ne, 4), b_full,
                                 kv_w1[0:1], kv_w2, kn_w.reshape(1, D),
                                 kn_b.reshape(1, D), lo, ne, be)
        a2, axf = _sc_scatter(dst, att, ax4.reshape(4 * ne), z128, zflat,
                              lo, ne)
        parts.append((a2, axf.reshape(2, FLAT)[:, :4 * N].reshape(2, N, 4)))

    x_out, h_out = _final(
        parts[0][0], parts[1][0], parts[0][1], parts[1][1], xn8, H,
        phix_w1, phix_w2.reshape(1, D),
        phih_w1, phih_b1.reshape(1, D), phih_w2, phih_b2.reshape(1, D))
    return (x_out, h_out)


# speedup vs baseline: 1.0077x; 1.0077x over previous
import dataclasses
import functools

import jax
import jax.numpy as jnp
from jax import lax
from jax.experimental import pallas as pl
from jax.experimental.pallas import tpu as pltpu
from jax.experimental.pallas import tpu_sc as plsc

N = 10000
E = 320000
D = 128
NG = 64
G8 = 16

NC = 2
NS = 16
L = 16
NW = NC * NS
C = 80
ZR = 624
FLAT = 40960

BE = 5000
BN = 2000

SPLITS = ((0, 163840, 5120), (163840, 156160, 2560))


def _sc_compiler_params():
    cp = pltpu.CompilerParams()
    if "needs_layout_passes" in pltpu.CompilerParams.__dataclass_fields__:
        cp = dataclasses.replace(cp, needs_layout_passes=False)
    return cp


def _silu(x):
    return x * jax.nn.sigmoid(x)


def _ln(x, w, b):
    m = jnp.mean(x, axis=-1, keepdims=True)
    v = jnp.mean((x - m) ** 2, axis=-1, keepdims=True)
    return (x - m) / jnp.sqrt(v + 1e-5) * w + b


def _node_stats_body(b2_ref, br_ref, x8_ref, e3_ref, xn8_ref):
    x8 = x8_ref[...]
    norm = jnp.sqrt(jnp.sum(x8 * x8, axis=-1, keepdims=True))
    gcol = lax.broadcasted_iota(jnp.int32, (NG, 1), 0)
    ohT = (gcol == br_ref[...]).astype(jnp.float32)
    nc = jnp.concatenate([norm, jnp.ones_like(norm)], axis=1)
    sums = jnp.dot(ohT, nc, preferred_element_type=jnp.float32)
    mg = sums[:, 0:1] / jnp.maximum(sums[:, 1:2], 1.0)
    grow = lax.broadcasted_iota(jnp.int32, (1, NG), 1)
    oh = (b2_ref[...] == grow).astype(jnp.float32)
    mn = jnp.dot(oh, mg, preferred_element_type=jnp.float32)
    xn8_ref[...] = e3_ref[0, 0] * x8 / (mn + 1e-5)


def _node_stats(batch2, batchr, x8, e3w):
    return pl.pallas_call(
        _node_stats_body,
        out_shape=jax.ShapeDtypeStruct((N, G8), jnp.float32),
    )(batch2, batchr, x8, e3w)


def _node_feat_body(h_ref, qw1_ref, qw2_ref, qnw_ref, qnb_ref,
                    gw1_ref, gb1_ref, gw2r_ref, gb2_ref, kvw1h_ref,
                    gate_ref, qn_ref, hkv_ref):
    h = h_ref[...]
    q = jnp.dot(_silu(jnp.dot(h, qw1_ref[...],
                              preferred_element_type=jnp.float32)),
                qw2_ref[...], preferred_element_type=jnp.float32)
    qn_ref[...] = _ln(q, qnw_ref[...], qnb_ref[...])
    g1 = _silu(jnp.dot(h, gw1_ref[...],
                       preferred_element_type=jnp.float32) + gb1_ref[...])
    gate_ref[...] = jax.nn.sigmoid(
        jnp.sum(g1 * gw2r_ref[...], axis=-1, keepdims=True) + gb2_ref[...])
    hkv_ref[...] = jnp.dot(h, kvw1h_ref[...],
                           preferred_element_type=jnp.float32)


def _node_feats(h, qw1, qw2, qnw, qnb, gw1, gb1, gw2r, gb2, kvw1h):
    nblk = N // BN
    row_spec = lambda shp: pl.BlockSpec(shp, lambda i: (i, 0))
    full = lambda shp: pl.BlockSpec(shp, lambda i: (0, 0))
    return pl.pallas_call(
        _node_feat_body,
        grid=(nblk,),
        in_specs=[row_spec((BN, D)),
                  full((D, D)), full((D, D)), full((1, D)), full((1, D)),
                  full((D, D)), full((1, D)), full((1, D)), full((1, 1)),
                  full((D, 2 * D))],
        out_specs=[row_spec((BN, 1)), row_spec((BN, D)),
                   row_spec((BN, 2 * D))],
        out_shape=[jax.ShapeDtypeStruct((N, 1), jnp.float32),
                   jax.ShapeDtypeStruct((N, D), jnp.float32),
                   jax.ShapeDtypeStruct((N, 2 * D), jnp.float32)],
    )(h, qw1, qw2, qnw, qnb, gw1, gb1, gw2r, gb2, kvw1h)


def _pb_body(ze_ref, pbw1_ref, pbb1_ref, pbw2r_ref, pbb2_ref, b_ref):
    zew = _silu(jnp.dot(ze_ref[...], pbw1_ref[...],
                        preferred_element_type=jnp.float32) + pbb1_ref[...])
    b_ref[...] = jnp.sum(zew * pbw2r_ref[...], axis=-1,
                         keepdims=True) + pbb2_ref[...]


def _pb_compute(ze, pbw1, pbb1, pbw2r, pbb2):
    nblk = E // BE
    row_spec = lambda shp: pl.BlockSpec(shp, lambda i: (i, 0))
    full = lambda shp: pl.BlockSpec(shp, lambda i: (0, 0))
    return pl.pallas_call(
        _pb_body,
        grid=(nblk,),
        in_specs=[row_spec((BE, D)), full((D, 4 * D)), full((1, 4 * D)),
                  full((1, 4 * D)), full((1, 1))],
        out_specs=row_spec((BE, 1)),
        out_shape=jax.ShapeDtypeStruct((E, 1), jnp.float32),
    )(ze, pbw1, pbb1, pbw2r, pbb2)


def _sc_gather_body(lo, nchunk, src_hbm, dst_hbm, hkv_hbm, qn_hbm,
                    tab4_hbm, hkvs_out, qnd_out, rel4_out,
                    sidx, didx, hbuf0, hbuf1, qbuf0, qbuf1, r4b0, r4b1,
                    tab4, s1, s2, s3, s4, so0, so1):
    ew = nchunk * C
    wid = lax.axis_index("s") * NC + lax.axis_index("c")
    base = wid * ew
    pltpu.sync_copy(tab4_hbm, tab4)
    pltpu.sync_copy(src_hbm.at[pl.ds(lo + base, ew)], sidx)
    pltpu.sync_copy(dst_hbm.at[pl.ds(lo + base, ew)], didx)

    def reg_work(g, r4b):
        @pl.loop(0, C // L)
        def _(j):
            sl = pl.ds(g * C + j * L, L)
            sv4 = sidx[sl] * 4
            dv4 = didx[sl] * 4
            ex = plsc.load_gather(tab4, [sv4]) - plsc.load_gather(tab4, [dv4])
            ey = (plsc.load_gather(tab4, [sv4 + 1])
                  - plsc.load_gather(tab4, [dv4 + 1]))
            ez = (plsc.load_gather(tab4, [sv4 + 2])
                  - plsc.load_gather(tab4, [dv4 + 2]))
            gg = plsc.load_gather(tab4, [sv4 + 3])
            pos = (lax.iota(jnp.int32, L) + j * L) * 4
            plsc.store_scatter(r4b, [pos], ex)
            plsc.store_scatter(r4b, [pos + 1], ey)
            plsc.store_scatter(r4b, [pos + 2], ez)
            plsc.store_scatter(r4b, [pos + 3], gg)

    def fire_gathers(g, hbuf, qbuf, sa, sb):
        ca = pltpu.async_copy(hkv_hbm.at[sidx.at[pl.ds(g * C, C)]], hbuf, sa)
        cb = pltpu.async_copy(qn_hbm.at[didx.at[pl.ds(g * C, C)]], qbuf, sb)
        return ca, cb

    def fire_outs(g, hbuf, qbuf, r4b, so):
        off = base + g * C
        o1 = pltpu.async_copy(hbuf, hkvs_out.at[pl.ds(off, C)], so)
        o2 = pltpu.async_copy(qbuf, qnd_out.at[pl.ds(off, C)], so)
        o3 = pltpu.async_copy(r4b, rel4_out.at[pl.ds(off * 4, 4 * C)], so)
        return o1, o2, o3

    def drain_outs(hbuf, qbuf, r4b, so):
        pltpu.make_async_copy(hbuf, hkvs_out.at[pl.ds(base, C)], so).wait()
        pltpu.make_async_copy(qbuf, qnd_out.at[pl.ds(base, C)], so).wait()
        pltpu.make_async_copy(r4b, rel4_out.at[pl.ds(base * 4, 4 * C)],
                              so).wait()

    @pl.loop(0, nchunk // 2)
    def _(p):
        g0 = 2 * p
        g1 = g0 + 1

        @pl.when(p > 0)
        def _():
            drain_outs(hbuf0, qbuf0, r4b0, so0)

        c1, c2 = fire_gathers(g0, hbuf0, qbuf0, s1, s2)

        @pl.when(p > 0)
        def _():
            drain_outs(hbuf1, qbuf1, r4b1, so1)

        c3, c4 = fire_gathers(g1, hbuf1, qbuf1, s3, s4)
        reg_work(g0, r4b0)
        reg_work(g1, r4b1)
        c1.wait()
        c2.wait()
        fire_outs(g0, hbuf0, qbuf0, r4b0, so0)
        c3.wait()
        c4.wait()
        fire_outs(g1, hbuf1, qbuf1, r4b1, so1)

    drain_outs(hbuf1, qbuf1, r4b1, so1)

    if nchunk % 2:
        g = nchunk - 1
        drain_outs(hbuf0, qbuf0, r4b0, so0)
        c1, c2 = fire_gathers(g, hbuf0, qbuf0, s1, s2)
        reg_work(g, r4b0)
        c1.wait()
        c2.wait()
        for o in fire_outs(g, hbuf0, qbuf0, r4b0, so0):
            o.wait()
    else:
        drain_outs(hbuf0, qbuf0, r4b0, so0)


def _sc_gather(src, dst, hkv, qn, tab4, lo, ne):
    ew = ne // NW
    assert ew % C == 0
    mesh = plsc.VectorSubcoreMesh(core_axis_name="c", subcore_axis_name="s",
                                  num_cores=NC, num_subcores=NS)
    f = functools.partial(
        pl.kernel,
        out_type=(jax.ShapeDtypeStruct((ne, 2 * D), jnp.float32),
                  jax.ShapeDtypeStruct((ne, D), jnp.float32),
                  jax.ShapeDtypeStruct((4 * ne,), jnp.float32)),
        mesh=mesh,
        scratch_types=[pltpu.VMEM((ew,), jnp.int32),
                       pltpu.VMEM((ew,), jnp.int32),
                       pltpu.VMEM((C, 2 * D), jnp.float32),
                       pltpu.VMEM((C, 2 * D), jnp.float32),
                       pltpu.VMEM((C, D), jnp.float32),
                       pltpu.VMEM((C, D), jnp.float32),
                       pltpu.VMEM((4 * C,), jnp.float32),
                       pltpu.VMEM((4 * C,), jnp.float32),
                       pltpu.VMEM((4 * N,), jnp.float32),
                       pltpu.SemaphoreType.DMA,
                       pltpu.SemaphoreType.DMA,
                       pltpu.SemaphoreType.DMA,
                       pltpu.SemaphoreType.DMA,
                       pltpu.SemaphoreType.DMA,
                       pltpu.SemaphoreType.DMA],
        compiler_params=_sc_compiler_params(),
    )(functools.partial(_sc_gather_body, lo, ew // C))
    return f(src, dst, hkv, qn, tab4)


def _edge_body(hkvs_ref, qnd_ref, rel4_ref, b_ref,
               kvw10_ref, kvw2_ref, knw_ref, knb_ref,
               att_ref, ax4_ref):
    rel4 = rel4_ref[...]
    rx = rel4[:, 0:1]
    ry = rel4[:, 1:2]
    rz = rel4[:, 2:3]
    gs = rel4[:, 3:4]
    rd = rx * rx + ry * ry + rz * rz
    inv = 1.0 / (1.0 + jnp.sqrt(rd + 1e-8))
    pre = hkvs_ref[...] + rd * kvw10_ref[...]
    kv = jnp.dot(_silu(pre), kvw2_ref[...],
                 preferred_element_type=jnp.float32)
    k = _ln(kv[:, :D], knw_ref[...], knb_ref[...])
    v = kv[:, D:]
    scores = jnp.sum(qnd_ref[...] * k, axis=-1, keepdims=True) * (
        1.0 / jnp.sqrt(jnp.float32(D))) + b_ref[...]
    alpha = gs * scores
    att_ref[...] = alpha * v
    w = alpha * inv
    ax4_ref[...] = jnp.concatenate(
        [w * rx, w * ry, w * rz, jnp.zeros_like(w)], axis=1)


def _edge_compute(hkvs, qnd, rel4, b, kvw10, kvw2, knw, knb, lo, ne, be):
    nblk = ne // be
    assert ne % be == 0 and be % 8 == 0 and lo % be == 0
    ob = lo // be
    row_spec = lambda shp: pl.BlockSpec(shp, lambda i: (i, 0))
    off_spec = lambda shp: pl.BlockSpec(shp, lambda i: (i + ob, 0))
    full = lambda shp: pl.BlockSpec(shp, lambda i: (0, 0))
    return pl.pallas_call(
        _edge_body,
        grid=(nblk,),
        in_specs=[row_spec((be, 2 * D)), row_spec((be, D)),
                  row_spec((be, 4)), off_spec((be, 1)),
                  full((1, 2 * D)), full((2 * D, 2 * D)),
                  full((1, D)), full((1, D))],
        out_specs=[row_spec((be, D)), row_spec((be, 4))],
        out_shape=[jax.ShapeDtypeStruct((ne, D), jnp.float32),
                   jax.ShapeDtypeStruct((ne, 4), jnp.float32)],
    )(hkvs, qnd, rel4, b, kvw10, kvw2, knw, knb)


def _sc_scatter_body(lo, nchunk, dst_hbm, att_hbm, ax4_hbm, z128_hbm, zflat_hbm,
                     a_out, axf_out,
                     didx0, didx1, ix0, iy0, iz0, ix1, iy1, iz1,
                     attb0, attb1, a4b0, a4b1, vx0, vy0, vz0, vx1, vy1, vz1,
                     a_sh, axf_sh, s1, s2, s3, s4, sw0, sw1):
    ew = nchunk * C
    cid = lax.axis_index("c")
    sid = lax.axis_index("s")
    wid = sid * NC + cid
    pltpu.sync_copy(z128_hbm, a_sh.at[pl.ds(sid * ZR, ZR)])

    @pl.when(sid == 0)
    def _():
        pltpu.sync_copy(z128_hbm.at[pl.ds(0, N - NS * ZR)],
                        a_sh.at[pl.ds(NS * ZR, N - NS * ZR)])
        pltpu.sync_copy(zflat_hbm, axf_sh)

    plsc.subcore_barrier()
    base = wid * ew

    def fire_loads(g, didx, attb, a4b, sa, sb):
        off = base + g * C
        pltpu.sync_copy(dst_hbm.at[pl.ds(lo + off, C)], didx)
        ca = pltpu.async_copy(att_hbm.at[pl.ds(off, C)], attb, sa)
        cb = pltpu.async_copy(ax4_hbm.at[pl.ds(off * 4, 4 * C)], a4b, sb)
        return ca, cb

    def reg_work(didx, a4b, ix, iy, iz, vx, vy, vz):
        @pl.loop(0, C // L)
        def _(j):
            sl = pl.ds(j * L, L)
            dv4 = didx[sl] * 4
            ix[sl] = dv4
            iy[sl] = dv4 + 1
            iz[sl] = dv4 + 2
            pos = (lax.iota(jnp.int32, L) + j * L) * 4
            vx[sl] = plsc.load_gather(a4b, [pos])
            vy[sl] = plsc.load_gather(a4b, [pos + 1])
            vz[sl] = plsc.load_gather(a4b, [pos + 2])

    def fire_scatters(didx, attb, ix, iy, iz, vx, vy, vz, sw):
        o1 = pltpu.async_copy(attb, a_sh.at[didx], sw, add=True)
        o2 = pltpu.async_copy(vx, axf_sh.at[ix], sw, add=True)
        o3 = pltpu.async_copy(vy, axf_sh.at[iy], sw, add=True)
        o4 = pltpu.async_copy(vz, axf_sh.at[iz], sw, add=True)
        return o1, o2, o3, o4

    def drain_scatters(didx, attb, ix, iy, iz, vx, vy, vz, sw):
        pltpu.make_async_copy(attb, a_sh.at[didx], sw).wait()
        pltpu.make_async_copy(vx, axf_sh.at[ix], sw).wait()
        pltpu.make_async_copy(vy, axf_sh.at[iy], sw).wait()
        pltpu.make_async_copy(vz, axf_sh.at[iz], sw).wait()

    @pl.loop(0, nchunk // 2)
    def _(p):
        g0 = 2 * p
        g1 = g0 + 1

        @pl.when(p > 0)
        def _():
            drain_scatters(didx0, attb0, ix0, iy0, iz0, vx0, vy0, vz0, sw0)

        c1, c2 = fire_loads(g0, didx0, attb0, a4b0, s1, s2)

        @pl.when(p > 0)
        def _():
            drain_scatters(didx1, attb1, ix1, iy1, iz1, vx1, vy1, vz1, sw1)

        c3, c4 = fire_loads(g1, didx1, attb1, a4b1, s3, s4)
        c1.wait()
        c2.wait()
        reg_work(didx0, a4b0, ix0, iy0, iz0, vx0, vy0, vz0)
        fire_scatters(didx0, attb0, ix0, iy0, iz0, vx0, vy0, vz0, sw0)
        c3.wait()
        c4.wait()
        reg_work(didx1, a4b1, ix1, iy1, iz1, vx1, vy1, vz1)
        fire_scatters(didx1, attb1, ix1, iy1, iz1, vx1, vy1, vz1, sw1)

    drain_scatters(didx1, attb1, ix1, iy1, iz1, vx1, vy1, vz1, sw1)

    if nchunk % 2:
        g = nchunk - 1
        drain_scatters(didx0, attb0, ix0, iy0, iz0, vx0, vy0, vz0, sw0)
        c1, c2 = fire_loads(g, didx0, attb0, a4b0, s1, s2)
        c1.wait()
        c2.wait()
        reg_work(didx0, a4b0, ix0, iy0, iz0, vx0, vy0, vz0)
        for o in fire_scatters(didx0, attb0, ix0, iy0, iz0, vx0, vy0, vz0,
                               sw0):
            o.wait()
    else:
        drain_scatters(didx0, attb0, ix0, iy0, iz0, vx0, vy0, vz0, sw0)

    plsc.subcore_barrier()
    row0 = cid * N + sid * ZR
    pltpu.sync_copy(a_sh.at[pl.ds(sid * ZR, ZR)], a_out.at[pl.ds(row0, ZR)])

    @pl.when(sid == 0)
    def _():
        pltpu.sync_copy(a_sh.at[pl.ds(NS * ZR, N - NS * ZR)],
                        a_out.at[pl.ds(cid * N + NS * ZR, N - NS * ZR)])
        pltpu.sync_copy(axf_sh, axf_out.at[pl.ds(cid * FLAT, FLAT)])


def _sc_scatter(dst, att, ax4, z128, zflat, lo, ne):
    ew = ne // NW
    assert ew % C == 0
    mesh = plsc.VectorSubcoreMesh(core_axis_name="c", subcore_axis_name="s",
                                  num_cores=NC, num_subcores=NS)
    cvec = lambda dt: pltpu.VMEM((C,), dt)
    f = functools.partial(
        pl.kernel,
        out_type=(jax.ShapeDtypeStruct((2 * N, D), jnp.float32),
                  jax.ShapeDtypeStruct((2 * FLAT,), jnp.float32)),
        mesh=mesh,
        scratch_types=[cvec(jnp.int32), cvec(jnp.int32),
                       cvec(jnp.int32), cvec(jnp.int32), cvec(jnp.int32),
                       cvec(jnp.int32), cvec(jnp.int32), cvec(jnp.int32),
                       pltpu.VMEM((C, D), jnp.float32),
                       pltpu.VMEM((C, D), jnp.float32),
                       pltpu.VMEM((4 * C,), jnp.float32),
                       pltpu.VMEM((4 * C,), jnp.float32),
                       cvec(jnp.float32), cvec(jnp.float32),
                       cvec(jnp.float32), cvec(jnp.float32),
                       cvec(jnp.float32), cvec(jnp.float32),
                       pltpu.VMEM_SHARED((N, D), jnp.float32),
                       pltpu.VMEM_SHARED((FLAT,), jnp.float32),
                       pltpu.SemaphoreType.DMA,
                       pltpu.SemaphoreType.DMA,
                       pltpu.SemaphoreType.DMA,
                       pltpu.SemaphoreType.DMA,
                       pltpu.SemaphoreType.DMA,
                       pltpu.SemaphoreType.DMA],
        compiler_params=_sc_compiler_params(),
    )(functools.partial(_sc_scatter_body, lo, ew // C))
    return f(dst, att, ax4, z128, zflat)


def _final_body(a2a_ref, a2b_ref, axfa_ref, axfb_ref, xn8_ref, h_ref,
                pxw1_ref, pxw2r_ref, phw1_ref, phb1_ref, phw2_ref, phb2_ref,
                xout_ref, hout_ref):
    a2 = a2a_ref[...] + a2b_ref[...]
    a = a2[0] + a2[1]
    axf = axfa_ref[...] + axfb_ref[...]
    axv = axf[0, :, 0:3] + axf[1, :, 0:3]
    phix = jnp.sum(_silu(jnp.dot(a, pxw1_ref[...],
                                 preferred_element_type=jnp.float32))
                   * pxw2r_ref[...], axis=-1, keepdims=True)
    xout_ref[...] = xn8_ref[...][:, 0:3] + phix * axv
    h = h_ref[...]
    t = _silu(jnp.dot(a * a * h, phw1_ref[...],
                      preferred_element_type=jnp.float32) + phb1_ref[...])
    hout_ref[...] = h + jnp.dot(t, phw2_ref[...],
                                preferred_element_type=jnp.float32) + phb2_ref[...]


def _final(a2a, a2b, axfa, axfb, xn8, h, pxw1, pxw2r, phw1, phb1, phw2,
           phb2):
    nblk = N // BN
    rs3 = lambda shp: pl.BlockSpec(shp, lambda i: (0, i, 0))
    rs2 = lambda shp: pl.BlockSpec(shp, lambda i: (i, 0))
    full = lambda shp: pl.BlockSpec(shp, lambda i: (0, 0))
    return pl.pallas_call(
        _final_body,
        grid=(nblk,),
        in_specs=[rs3((2, BN, D)), rs3((2, BN, D)),
                  rs3((2, BN, 4)), rs3((2, BN, 4)),
                  rs2((BN, G8)), rs2((BN, D)),
                  full((D, D)), full((1, D)), full((D, D)), full((1, D)),
                  full((D, D)), full((1, D))],
        out_specs=[rs2((BN, 3)), rs2((BN, D))],
        out_shape=[jax.ShapeDtypeStruct((N, 3), jnp.float32),
                   jax.ShapeDtypeStruct((N, D), jnp.float32)],
    )(a2a.reshape(2, N, D), a2b.reshape(2, N, D), axfa, axfb, xn8, h,
      pxw1, pxw2r, phw1, phb1, phw2, phb2)


def kernel(batch, X, H, E_idx, ZE, e3_w, kv_w1, kv_w2, q_w1, q_w2, pb_w1,
           pb_b1, pb_w2, pb_b2, gate_w1, gate_b1, gate_w2, gate_b2, phih_w1,
           phih_b1, phih_w2, phih_b2, phix_w1, phix_w2, qn_w, qn_b, kn_w,
           kn_b):
    batch2 = batch.reshape(N, 1).astype(jnp.int32)
    batchr = batch.reshape(1, N).astype(jnp.int32)
    x8 = jnp.concatenate([X, jnp.zeros((N, G8 - 3), jnp.float32)], axis=1)
    src = E_idx[0].astype(jnp.int32)
    dst = E_idx[1].astype(jnp.int32)

    xn8 = _node_stats(batch2, batchr, x8, e3_w)
    gate, qn, hkv = _node_feats(
        H, q_w1, q_w2, qn_w.reshape(1, D), qn_b.reshape(1, D),
        gate_w1, gate_b1.reshape(1, D), gate_w2.reshape(1, D),
        gate_b2.reshape(1, 1), kv_w1[1:])

    z128 = jnp.zeros((ZR, D), jnp.float32)
    zflat = jnp.zeros((FLAT,), jnp.float32)
    tab4 = jnp.concatenate([xn8[:, 0:3], gate], axis=1).reshape(4 * N)

    b_full = _pb_compute(ZE, pb_w1, pb_b1.reshape(1, 4 * D),
                         pb_w2.reshape(1, 4 * D), pb_b2.reshape(1, 1))

    gouts = [_sc_gather(src, dst, hkv, qn, tab4, lo, ne)
             for lo, ne, be in SPLITS]

    parts = []
    for (lo, ne, be), g in zip(SPLITS, gouts):
        hkvs, qnd, rel4 = g
        att, ax4 = _edge_compute(hkvs, qnd, rel4.reshape(ne, 4), b_full,
                                 kv_w1[0:1], kv_w2, kn_w.reshape(1, D),
                                 kn_b.reshape(1, D), lo, ne, be)
        a2, axf = _sc_scatter(dst, att, ax4.reshape(4 * ne), z128, zflat,
                              lo, ne)
        parts.append((a2, axf.reshape(2, FLAT)[:, :4 * N].reshape(2, N, 4)))

    x_out, h_out = _final(
        parts[0][0], parts[1][0], parts[0][1], parts[1][1], xn8, H,
        phix_w1, phix_w2.reshape(1, D),
        phih_w1, phih_b1.reshape(1, D), phih_w2, phih_b2.reshape(1, D))
    return (x_out, h_out)
